# Initial kernel scaffold; baseline (speedup 1.0000x reference)
import jax
import jax.numpy as jnp
from jax.experimental import pallas as pl


def kernel(x, edge_index, batch, edge_attr, gcn_params, mlp_params, out_W, out_b):
    raise NotImplementedError("write your pallas kernel here")


# speedup vs baseline: 2.0955x; 2.0955x over previous
import functools

import jax
import jax.numpy as jnp
from jax import lax
from jax.experimental import pallas as pl
from jax.experimental.pallas import tpu as pltpu
from jax.experimental.pallas import tpu_sc as plsc

N = 10000
E = 320000
NUM_GRAPHS = 64

NTILES = 32
EDGES_PER_TILE = E // NTILES
BATCH = 80
NBATCH = EDGES_PER_TILE // BATCH
NPAD = 10240
NHALF = NPAD // 2
ACC_ROWS = NHALF + 8
ROWS_PER_TILE = NHALF // 16
ZROWS = 160


@functools.lru_cache(maxsize=None)
def _make_sc_segsum(P, Ck):
    mesh = plsc.VectorSubcoreMesh(core_axis_name="c", subcore_axis_name="s")
    gpe = Ck // 16

    @functools.partial(
        pl.kernel,
        mesh=mesh,
        out_type=jax.ShapeDtypeStruct((2, P, NPAD, Ck), jnp.float32),
        scratch_types=[
            pltpu.VMEM((EDGES_PER_TILE,), jnp.int32),
            pltpu.VMEM((EDGES_PER_TILE,), jnp.int32),
            pltpu.VMEM((EDGES_PER_TILE,), jnp.float32),
            pltpu.VMEM((BATCH, Ck), jnp.float32),
            pltpu.VMEM((BATCH,), jnp.int32),
            pltpu.VMEM((ZROWS, Ck), jnp.float32),
            pltpu.VMEM((ZROWS, Ck), jnp.float32),
            pltpu.VMEM_SHARED((ACC_ROWS, Ck), jnp.float32),
            pltpu.SemaphoreType.DMA,
        ],
    )
    def seg_sum(h_hbm, src_hbm, dst_hbm, attr_hbm, out_hbm,
                src_v, dst_v, attr_v, rows_v, dstb_v, zero_v, stage_v,
                accum, sem):
        cid = lax.axis_index("c")
        sid = lax.axis_index("s")
        tile_edge0 = (cid * 16 + sid) * EDGES_PER_TILE
        row0 = sid * ROWS_PER_TILE

        pltpu.sync_copy(src_hbm.at[pl.ds(tile_edge0, EDGES_PER_TILE)], src_v)
        pltpu.sync_copy(dst_hbm.at[pl.ds(tile_edge0, EDGES_PER_TILE)], dst_v)
        pltpu.sync_copy(attr_hbm.at[pl.ds(tile_edge0, EDGES_PER_TILE)], attr_v)

        z16 = jnp.zeros((16,), jnp.float32)

        def zrow(r, _):
            for g in range(gpe):
                zero_v[r, pl.ds(g * 16, 16)] = z16
            return 0

        lax.fori_loop(0, ZROWS, zrow, 0)

        for p in range(P):
            for n in range(2):
                for k in range(ROWS_PER_TILE // ZROWS):
                    pltpu.sync_copy(
                        zero_v, accum.at[pl.ds(row0 + k * ZROWS, ZROWS)]
                    )
                plsc.subcore_barrier()

                def batch_body(b, _):
                    e0 = b * BATCH
                    pltpu.async_copy(
                        h_hbm.at[p].at[src_v.at[pl.ds(e0, BATCH)]], rows_v, sem
                    ).wait()
                    for g in range(BATCH // 16):
                        d16 = dst_v[pl.ds(e0 + g * 16, 16)] - (n * NHALF)
                        ok = (d16 >= 0) & (d16 < NHALF)
                        dstb_v[pl.ds(g * 16, 16)] = jnp.where(ok, d16, NHALF)

                    def grp_body(g16, _):
                        w16 = attr_v[pl.ds(e0 + g16 * 16, 16)]
                        for ep in range(16):
                            w = jnp.full((16,), w16[ep], jnp.float32)
                            e = g16 * 16 + ep
                            for g in range(gpe):
                                sl = pl.ds(g * 16, 16)
                                rows_v[e, sl] = rows_v[e, sl] * w
                        return 0

                    lax.fori_loop(0, BATCH // 16, grp_body, 0)

                    pltpu.sync_copy(rows_v, accum.at[dstb_v], add=True)
                    return 0

                lax.fori_loop(0, NBATCH, batch_body, 0)
                plsc.subcore_barrier()

                for k in range(ROWS_PER_TILE // ZROWS):
                    r = row0 + k * ZROWS
                    pltpu.sync_copy(accum.at[pl.ds(r, ZROWS)], stage_v)
                    pltpu.sync_copy(
                        stage_v,
                        out_hbm.at[cid, p].at[pl.ds(n * NHALF + r, ZROWS)],
                    )
                plsc.subcore_barrier()

    return seg_sum


ROWB = 1000


def _dotT(a, w):
    return lax.dot_general(a, w, (((1,), (1,)), ((), ())),
                           precision=lax.Precision.HIGHEST,
                           preferred_element_type=jnp.float32)


def _dotT16(a, w):
    return lax.dot_general(a.astype(jnp.bfloat16), w.astype(jnp.bfloat16),
                           (((1,), (1,)), ((), ())),
                           preferred_element_type=jnp.float32)


def _gcn_tc_body(Pa, Ck, Pin, Ckin, Pout, Cko,
                 part_ref, h_ref, wrel_ref, wroot_ref, b_ref, out_ref):
    for po in range(Pout):
        acc = jnp.broadcast_to(b_ref[0, pl.ds(po * Cko, Cko)], (ROWB, Cko))
        for p in range(Pa):
            agg = part_ref[0, p] + part_ref[1, p]
            wslice = wrel_ref[pl.ds(po * Cko, Cko), pl.ds(p * Ck, Ck)]
            acc = acc + _dotT16(agg, wslice)
        for pi in range(Pin):
            wslice = wroot_ref[pl.ds(po * Cko, Cko), pl.ds(pi * Ckin, Ckin)]
            acc = acc + _dotT16(h_ref[pi], wslice)
        out_ref[po] = jnp.maximum(acc, 0.0)


@functools.lru_cache(maxsize=None)
def _make_gcn_tc(Pa, Ck, Pin, Ckin, Pout, Cko):
    body = functools.partial(_gcn_tc_body, Pa, Ck, Pin, Ckin, Pout, Cko)
    grid = N // ROWB
    return pl.pallas_call(
        body,
        grid=(grid,),
        in_specs=[
            pl.BlockSpec((2, Pa, ROWB, Ck), lambda i: (0, 0, i, 0)),
            pl.BlockSpec((Pin, ROWB, Ckin), lambda i: (0, i, 0)),
            pl.BlockSpec((Pout * Cko, Pa * Ck), lambda i: (0, 0)),
            pl.BlockSpec((Pout * Cko, Pin * Ckin), lambda i: (0, 0)),
            pl.BlockSpec((1, Pout * Cko), lambda i: (0, 0)),
        ],
        out_specs=pl.BlockSpec((Pout, ROWB, Cko), lambda i: (0, i, 0)),
        out_shape=jax.ShapeDtypeStruct((Pout, N, Cko), jnp.float32),
    )


def _pool_mlp_body(mlp_shapes, h_ref, batch_ref, w1, b1, w2, b2, w3, b3,
                   wo, bo, out_ref, acc_ref, cnt_ref):
    i = pl.program_id(0)
    P = h_ref.shape[0]
    hcat = jnp.concatenate([h_ref[p] for p in range(P)], axis=1)
    gids = jax.lax.broadcasted_iota(jnp.int32, (ROWB, NUM_GRAPHS), 1)
    onehot = (batch_ref[...] == gids).astype(jnp.float32)
    psum = lax.dot_general(onehot, hcat, (((0,), (0,)), ((), ())),
                           preferred_element_type=jnp.float32)
    pcnt = jnp.sum(onehot, axis=0)[:, None]

    @pl.when(i == 0)
    def _init():
        acc_ref[...] = psum
        cnt_ref[...] = pcnt

    @pl.when(i > 0)
    def _acc():
        acc_ref[...] = acc_ref[...] + psum
        cnt_ref[...] = cnt_ref[...] + pcnt

    @pl.when(i == pl.num_programs(0) - 1)
    def _head():
        g = acc_ref[...] / jnp.maximum(cnt_ref[...], 1.0)
        g = jnp.maximum(_dotT16(g, w1[...]) + b1[0], 0.0)
        g = jnp.maximum(_dotT16(g, w2[...]) + b2[0], 0.0)
        g = jnp.maximum(_dotT16(g, w3[...]) + b3[0], 0.0)
        gb = g.astype(jnp.bfloat16).astype(jnp.float32)
        wb = wo[...].astype(jnp.bfloat16).astype(jnp.float32)
        out_ref[...] = jnp.sum(gb * wb, axis=1, keepdims=True) + bo[...]


@functools.lru_cache(maxsize=None)
def _make_pool_mlp(P, Ck, d1i, d1o, d2o, d3o):
    body = functools.partial(_pool_mlp_body, None)
    grid = N // ROWB
    C = P * Ck
    return pl.pallas_call(
        body,
        grid=(grid,),
        in_specs=[
            pl.BlockSpec((P, ROWB, Ck), lambda i: (0, i, 0)),
            pl.BlockSpec((ROWB, 1), lambda i: (i, 0)),
            pl.BlockSpec((d1o, d1i), lambda i: (0, 0)),
            pl.BlockSpec((1, d1o), lambda i: (0, 0)),
            pl.BlockSpec((d2o, d1o), lambda i: (0, 0)),
            pl.BlockSpec((1, d2o), lambda i: (0, 0)),
            pl.BlockSpec((d3o, d2o), lambda i: (0, 0)),
            pl.BlockSpec((1, d3o), lambda i: (0, 0)),
            pl.BlockSpec((1, d3o), lambda i: (0, 0)),
            pl.BlockSpec((1, 1), lambda i: (0, 0)),
        ],
        out_specs=pl.BlockSpec((NUM_GRAPHS, 1), lambda i: (0, 0)),
        out_shape=jax.ShapeDtypeStruct((NUM_GRAPHS, 1), jnp.float32),
        scratch_shapes=[
            pltpu.VMEM((NUM_GRAPHS, C), jnp.float32),
            pltpu.VMEM((NUM_GRAPHS, 1), jnp.float32),
        ],
    )


def _padded(c):
    return max(128, c)


def kernel(x, edge_index, batch, edge_attr, gcn_params, mlp_params, out_W, out_b):
    src = edge_index[0]
    dst = edge_index[1]

    xp = jnp.pad(x, ((0, 0), (0, 124)))
    h = xp.reshape(1, N, 128)
    Pin, Ckin = 1, 128

    for li, (W_rel, b_rel, W_root) in enumerate(gcn_params):
        cout, cin = W_rel.shape
        cin_p, cout_p = _padded(cin), _padded(cout)
        W_rel = jnp.pad(W_rel, ((0, cout_p - cout), (0, cin_p - cin)))
        W_root = jnp.pad(W_root, ((0, cout_p - cout), (0, cin_p - cin)))
        b_rel = jnp.pad(b_rel, (0, cout_p - cout))
        Pa, Ck = Pin, Ckin
        Pout, Cko = cout_p // 128, 128

        part = _make_sc_segsum(Pa, Ck)(h, src, dst, edge_attr)
        h = _make_gcn_tc(Pa, Ck, Pin, Ckin, Pout, Cko)(
            part, h, W_rel, W_root, b_rel.reshape(1, -1)
        )
        Pin, Ckin = Pout, Cko

    (W1, b1), (W2, b2), (W3, b3) = mlp_params
    out = _make_pool_mlp(Pin, Ckin, W1.shape[1], W1.shape[0],
                         W2.shape[0], W3.shape[0])(
        h, batch.reshape(N, 1).astype(jnp.int32),
        W1, b1.reshape(1, -1), W2, b2.reshape(1, -1),
        W3, b3.reshape(1, -1), out_W, out_b.reshape(1, 1),
    )
    return out


# speedup vs baseline: 2.0955x; 1.0000x over previous
import functools

import jax
import jax.numpy as jnp
from jax import lax
from jax.experimental import pallas as pl
from jax.experimental.pallas import tpu as pltpu
from jax.experimental.pallas import tpu_sc as plsc

N = 10000
E = 320000
NUM_GRAPHS = 64

NTILES = 32
EDGES_PER_TILE = E // NTILES
BATCH = 80
NBATCH = EDGES_PER_TILE // BATCH
NPAD = 10240
NHALF = NPAD // 2
ACC_ROWS = NHALF + 8
ROWS_PER_TILE = NHALF // 16
ZROWS = 160


@functools.lru_cache(maxsize=None)
def _make_sc_segsum(P, Ck):
    mesh = plsc.VectorSubcoreMesh(core_axis_name="c", subcore_axis_name="s")
    gpe = Ck // 16

    @functools.partial(
        pl.kernel,
        mesh=mesh,
        out_type=jax.ShapeDtypeStruct((2, P, NPAD, Ck), jnp.float32),
        scratch_types=[
            pltpu.VMEM((EDGES_PER_TILE,), jnp.int32),
            pltpu.VMEM((EDGES_PER_TILE,), jnp.int32),
            pltpu.VMEM((EDGES_PER_TILE,), jnp.float32),
            pltpu.VMEM((BATCH, Ck), jnp.float32),
            pltpu.VMEM((BATCH,), jnp.int32),
            pltpu.VMEM((ZROWS, Ck), jnp.float32),
            pltpu.VMEM((ZROWS, Ck), jnp.float32),
            pltpu.VMEM_SHARED((ACC_ROWS, Ck), jnp.float32),
            pltpu.SemaphoreType.DMA,
        ],
    )
    def seg_sum(h_hbm, src_hbm, dst_hbm, attr_hbm, out_hbm,
                src_v, dst_v, attr_v, rows_v, dstb_v, zero_v, stage_v,
                accum, sem):
        cid = lax.axis_index("c")
        sid = lax.axis_index("s")
        tile_edge0 = (cid * 16 + sid) * EDGES_PER_TILE
        row0 = sid * ROWS_PER_TILE

        pltpu.sync_copy(src_hbm.at[pl.ds(tile_edge0, EDGES_PER_TILE)], src_v)
        pltpu.sync_copy(dst_hbm.at[pl.ds(tile_edge0, EDGES_PER_TILE)], dst_v)
        pltpu.sync_copy(attr_hbm.at[pl.ds(tile_edge0, EDGES_PER_TILE)], attr_v)

        z16 = jnp.zeros((16,), jnp.float32)

        def zrow(r, _):
            for g in range(gpe):
                zero_v[r, pl.ds(g * 16, 16)] = z16
            return 0

        lax.fori_loop(0, ZROWS, zrow, 0)

        for p in range(P):
            for n in range(2):
                for k in range(ROWS_PER_TILE // ZROWS):
                    pltpu.sync_copy(
                        zero_v, accum.at[pl.ds(row0 + k * ZROWS, ZROWS)]
                    )
                plsc.subcore_barrier()

                def batch_body(b, _):
                    e0 = b * BATCH
                    pltpu.async_copy(
                        h_hbm.at[p].at[src_v.at[pl.ds(e0, BATCH)]], rows_v, sem
                    ).wait()
                    for g in range(BATCH // 16):
                        d16 = dst_v[pl.ds(e0 + g * 16, 16)] - (n * NHALF)
                        ok = (d16 >= 0) & (d16 < NHALF)
                        dstb_v[pl.ds(g * 16, 16)] = jnp.where(ok, d16, NHALF)

                    def grp_body(g16, _):
                        w16 = attr_v[pl.ds(e0 + g16 * 16, 16)]
                        for ep in range(16):
                            w = jnp.full((16,), w16[ep], jnp.float32)
                            e = g16 * 16 + ep
                            for g in range(gpe):
                                sl = pl.ds(g * 16, 16)
                                rows_v[e, sl] = rows_v[e, sl] * w
                        return 0

                    lax.fori_loop(0, BATCH // 16, grp_body, 0)

                    pltpu.sync_copy(rows_v, accum.at[dstb_v], add=True)
                    return 0

                lax.fori_loop(0, NBATCH, batch_body, 0)
                plsc.subcore_barrier()

                for k in range(ROWS_PER_TILE // ZROWS):
                    r = row0 + k * ZROWS
                    pltpu.sync_copy(accum.at[pl.ds(r, ZROWS)], stage_v)
                    pltpu.sync_copy(
                        stage_v,
                        out_hbm.at[cid, p].at[pl.ds(n * NHALF + r, ZROWS)],
                    )
                plsc.subcore_barrier()

    return seg_sum


ROWB = 1000


def _dotT(a, w):
    return lax.dot_general(a, w, (((1,), (1,)), ((), ())),
                           precision=lax.Precision.HIGHEST,
                           preferred_element_type=jnp.float32)


def _dotT16(a, w):
    return lax.dot_general(a.astype(jnp.bfloat16), w.astype(jnp.bfloat16),
                           (((1,), (1,)), ((), ())),
                           preferred_element_type=jnp.float32)


def _gcn_tc_body(Pa, Ck, Pin, Ckin, Pout, Cko,
                 part_ref, h_ref, wrel_ref, wroot_ref, b_ref, out_ref):
    for po in range(Pout):
        acc = jnp.broadcast_to(b_ref[0, pl.ds(po * Cko, Cko)], (ROWB, Cko))
        for p in range(Pa):
            agg = part_ref[0, p] + part_ref[1, p]
            wslice = wrel_ref[pl.ds(po * Cko, Cko), pl.ds(p * Ck, Ck)]
            acc = acc + _dotT16(agg, wslice)
        for pi in range(Pin):
            wslice = wroot_ref[pl.ds(po * Cko, Cko), pl.ds(pi * Ckin, Ckin)]
            acc = acc + _dotT16(h_ref[pi], wslice)
        out_ref[po] = jnp.maximum(acc, 0.0)


@functools.lru_cache(maxsize=None)
def _make_gcn_tc(Pa, Ck, Pin, Ckin, Pout, Cko):
    body = functools.partial(_gcn_tc_body, Pa, Ck, Pin, Ckin, Pout, Cko)
    grid = N // ROWB
    return pl.pallas_call(
        body,
        grid=(grid,),
        in_specs=[
            pl.BlockSpec((2, Pa, ROWB, Ck), lambda i: (0, 0, i, 0)),
            pl.BlockSpec((Pin, ROWB, Ckin), lambda i: (0, i, 0)),
            pl.BlockSpec((Pout * Cko, Pa * Ck), lambda i: (0, 0)),
            pl.BlockSpec((Pout * Cko, Pin * Ckin), lambda i: (0, 0)),
            pl.BlockSpec((1, Pout * Cko), lambda i: (0, 0)),
        ],
        out_specs=pl.BlockSpec((Pout, ROWB, Cko), lambda i: (0, i, 0)),
        out_shape=jax.ShapeDtypeStruct((Pout, N, Cko), jnp.float32),
    )


def _pool_mlp_body(mlp_shapes, h_ref, batch_ref, w1, b1, w2, b2, w3, b3,
                   wo, out_ref, acc_ref, cnt_ref):
    i = pl.program_id(0)
    P = h_ref.shape[0]
    hcat = jnp.concatenate([h_ref[p] for p in range(P)], axis=1)
    gids = jax.lax.broadcasted_iota(jnp.int32, (ROWB, NUM_GRAPHS), 1)
    onehot = (batch_ref[...] == gids).astype(jnp.float32)
    psum = lax.dot_general(onehot, hcat, (((0,), (0,)), ((), ())),
                           preferred_element_type=jnp.float32)
    pcnt = jnp.sum(onehot, axis=0)[:, None]

    @pl.when(i == 0)
    def _init():
        acc_ref[...] = psum
        cnt_ref[...] = pcnt

    @pl.when(i > 0)
    def _acc():
        acc_ref[...] = acc_ref[...] + psum
        cnt_ref[...] = cnt_ref[...] + pcnt

    @pl.when(i == pl.num_programs(0) - 1)
    def _head():
        g = acc_ref[...] / jnp.maximum(cnt_ref[...], 1.0)
        g = jnp.maximum(_dotT16(g, w1[...]) + b1[0], 0.0)
        g = jnp.maximum(_dotT16(g, w2[...]) + b2[0], 0.0)
        g = jnp.maximum(_dotT16(g, w3[...]) + b3[0], 0.0)
        out_ref[...] = _dotT16(g, wo[...])


@functools.lru_cache(maxsize=None)
def _make_pool_mlp(P, Ck, d1i, d1o, d2o, d3o):
    body = functools.partial(_pool_mlp_body, None)
    grid = N // ROWB
    C = P * Ck
    return pl.pallas_call(
        body,
        grid=(grid,),
        in_specs=[
            pl.BlockSpec((P, ROWB, Ck), lambda i: (0, i, 0)),
            pl.BlockSpec((ROWB, 1), lambda i: (i, 0)),
            pl.BlockSpec((d1o, d1i), lambda i: (0, 0)),
            pl.BlockSpec((1, d1o), lambda i: (0, 0)),
            pl.BlockSpec((d2o, d1o), lambda i: (0, 0)),
            pl.BlockSpec((1, d2o), lambda i: (0, 0)),
            pl.BlockSpec((d3o, d2o), lambda i: (0, 0)),
            pl.BlockSpec((1, d3o), lambda i: (0, 0)),
            pl.BlockSpec((128, d3o), lambda i: (0, 0)),
        ],
        out_specs=pl.BlockSpec((NUM_GRAPHS, 128), lambda i: (0, 0)),
        out_shape=jax.ShapeDtypeStruct((NUM_GRAPHS, 128), jnp.float32),
        scratch_shapes=[
            pltpu.VMEM((NUM_GRAPHS, C), jnp.float32),
            pltpu.VMEM((NUM_GRAPHS, 1), jnp.float32),
        ],
    )


def _padded(c):
    return max(128, c)


def kernel(x, edge_index, batch, edge_attr, gcn_params, mlp_params, out_W, out_b):
    src = edge_index[0]
    dst = edge_index[1]

    xp = jnp.pad(x, ((0, 0), (0, 124)))
    h = xp.reshape(1, N, 128)
    Pin, Ckin = 1, 128

    for li, (W_rel, b_rel, W_root) in enumerate(gcn_params):
        cout, cin = W_rel.shape
        cin_p, cout_p = _padded(cin), _padded(cout)
        W_rel = jnp.pad(W_rel, ((0, cout_p - cout), (0, cin_p - cin)))
        W_root = jnp.pad(W_root, ((0, cout_p - cout), (0, cin_p - cin)))
        b_rel = jnp.pad(b_rel, (0, cout_p - cout))
        Pa, Ck = Pin, Ckin
        Pout, Cko = cout_p // 128, 128

        part = _make_sc_segsum(Pa, Ck)(h, src, dst, edge_attr)
        h = _make_gcn_tc(Pa, Ck, Pin, Ckin, Pout, Cko)(
            part, h, W_rel, W_root, b_rel.reshape(1, -1)
        )
        Pin, Ckin = Pout, Cko

    (W1, b1), (W2, b2), (W3, b3) = mlp_params
    wo_p = jnp.pad(out_W, ((0, 127), (0, 0)))
    pooled = _make_pool_mlp(Pin, Ckin, W1.shape[1], W1.shape[0],
                            W2.shape[0], W3.shape[0])(
        h, batch.reshape(N, 1).astype(jnp.int32),
        W1, b1.reshape(1, -1), W2, b2.reshape(1, -1),
        W3, b3.reshape(1, -1), wo_p,
    )
    return pooled[:, :1] + out_b


# speedup vs baseline: 3.8681x; 1.8459x over previous
import functools

import jax
import jax.numpy as jnp
from jax import lax
from jax.experimental import pallas as pl
from jax.experimental.pallas import tpu as pltpu
from jax.experimental.pallas import tpu_sc as plsc

N = 10000
E = 320000
NUM_GRAPHS = 64

NTILES = 32
EDGES_PER_TILE = E // NTILES
BATCH = 80
NBATCH = EDGES_PER_TILE // BATCH
NPAD = 10240
NHALF = NPAD // 2
ACC_ROWS = NHALF + 8
ROWS_PER_TILE = NHALF // 16
ZROWS = 80


@functools.lru_cache(maxsize=None)
def _make_sc_segsum(P, Ck):
    mesh = plsc.VectorSubcoreMesh(core_axis_name="c", subcore_axis_name="s")
    gpe = Ck // 16

    @functools.partial(
        pl.kernel,
        mesh=mesh,
        out_type=jax.ShapeDtypeStruct((2, P, NPAD, Ck), jnp.float32),
        scratch_types=[
            pltpu.VMEM((EDGES_PER_TILE,), jnp.int32),
            pltpu.VMEM((EDGES_PER_TILE,), jnp.int32),
            pltpu.VMEM((EDGES_PER_TILE,), jnp.float32),
            pltpu.VMEM((BATCH, Ck), jnp.float32),
            pltpu.VMEM((BATCH, Ck), jnp.float32),
            pltpu.VMEM((BATCH,), jnp.int32),
            pltpu.VMEM((ZROWS, Ck), jnp.float32),
            pltpu.VMEM((ZROWS, Ck), jnp.float32),
            pltpu.VMEM_SHARED((ACC_ROWS, Ck), jnp.float32),
            pltpu.SemaphoreType.DMA,
            pltpu.SemaphoreType.DMA,
        ],
    )
    def seg_sum(h_hbm, src_hbm, dst_hbm, attr_hbm, out_hbm,
                src_v, dst_v, attr_v, rows0_v, rows1_v, dstb_v,
                zero_v, stage_v, accum, sem0, sem1):
        cid = lax.axis_index("c")
        sid = lax.axis_index("s")
        tile_edge0 = (cid * 16 + sid) * EDGES_PER_TILE
        row0 = sid * ROWS_PER_TILE

        pltpu.sync_copy(src_hbm.at[pl.ds(tile_edge0, EDGES_PER_TILE)], src_v)
        pltpu.sync_copy(dst_hbm.at[pl.ds(tile_edge0, EDGES_PER_TILE)], dst_v)
        pltpu.sync_copy(attr_hbm.at[pl.ds(tile_edge0, EDGES_PER_TILE)], attr_v)

        z16 = jnp.zeros((16,), jnp.float32)

        def zrow(r, _):
            for g in range(gpe):
                zero_v[r, pl.ds(g * 16, 16)] = z16
            return 0

        lax.fori_loop(0, ZROWS, zrow, 0)

        bufs = ((rows0_v, sem0), (rows1_v, sem1))

        def start_gather(p, b, buf, sem):
            pltpu.async_copy(
                h_hbm.at[p].at[src_v.at[pl.ds(b * BATCH, BATCH)]], buf, sem
            )

        def drain_gather(p, buf, sem):
            pltpu.make_async_copy(
                h_hbm.at[p].at[src_v.at[pl.ds(0, BATCH)]], buf, sem
            ).wait()

        for p in range(P):
            for n in range(2):
                for k in range(ROWS_PER_TILE // ZROWS):
                    pltpu.sync_copy(
                        zero_v, accum.at[pl.ds(row0 + k * ZROWS, ZROWS)]
                    )
                plsc.subcore_barrier()

                start_gather(p, 0, rows0_v, sem0)

                def pair_body(i, _):
                    for ph, (rbuf, sem) in enumerate(bufs):
                        b = i * 2 + ph
                        obuf, osem = bufs[1 - ph]

                        @pl.when(b < NBATCH)
                        def _do():
                            drain_gather(p, rbuf, sem)

                            @pl.when(b + 1 < NBATCH)
                            def _pre():
                                start_gather(p, b + 1, obuf, osem)

                            e0 = b * BATCH
                            for g in range(BATCH // 16):
                                d16 = dst_v[pl.ds(e0 + g * 16, 16)] - (n * NHALF)
                                ok = (d16 >= 0) & (d16 < NHALF)
                                dstb_v[pl.ds(g * 16, 16)] = jnp.where(
                                    ok, d16, NHALF)

                            def grp_body(g16, _):
                                w16 = attr_v[pl.ds(e0 + g16 * 16, 16)]
                                for ep in range(16):
                                    w = jnp.full((16,), w16[ep], jnp.float32)
                                    e = g16 * 16 + ep
                                    for g in range(gpe):
                                        sl = pl.ds(g * 16, 16)
                                        rbuf[e, sl] = rbuf[e, sl] * w
                                return 0

                            lax.fori_loop(0, BATCH // 16, grp_body, 0)

                            pltpu.sync_copy(rbuf, accum.at[dstb_v], add=True)

                    return 0

                lax.fori_loop(0, (NBATCH + 1) // 2, pair_body, 0)
                plsc.subcore_barrier()

                for k in range(ROWS_PER_TILE // ZROWS):
                    r = row0 + k * ZROWS
                    pltpu.sync_copy(accum.at[pl.ds(r, ZROWS)], stage_v)
                    pltpu.sync_copy(
                        stage_v,
                        out_hbm.at[cid, p].at[pl.ds(n * NHALF + r, ZROWS)],
                    )
                plsc.subcore_barrier()

    return seg_sum


ROWB = 1000


def _dotT(a, w):
    return lax.dot_general(a, w, (((1,), (1,)), ((), ())),
                           precision=lax.Precision.HIGHEST,
                           preferred_element_type=jnp.float32)


def _dotT16(a, w):
    return lax.dot_general(a.astype(jnp.bfloat16), w.astype(jnp.bfloat16),
                           (((1,), (1,)), ((), ())),
                           preferred_element_type=jnp.float32)


def _gcn_tc_body(Pa, Ck, Pin, Ckin, Pout, Cko,
                 part_ref, h_ref, wrel_ref, wroot_ref, b_ref, out_ref):
    for po in range(Pout):
        acc = jnp.broadcast_to(b_ref[0, pl.ds(po * Cko, Cko)], (ROWB, Cko))
        for p in range(Pa):
            agg = part_ref[0, p] + part_ref[1, p]
            wslice = wrel_ref[pl.ds(po * Cko, Cko), pl.ds(p * Ck, Ck)]
            acc = acc + _dotT16(agg, wslice)
        for pi in range(Pin):
            wslice = wroot_ref[pl.ds(po * Cko, Cko), pl.ds(pi * Ckin, Ckin)]
            acc = acc + _dotT16(h_ref[pi], wslice)
        out_ref[po] = jnp.maximum(acc, 0.0)


@functools.lru_cache(maxsize=None)
def _make_gcn_tc(Pa, Ck, Pin, Ckin, Pout, Cko):
    body = functools.partial(_gcn_tc_body, Pa, Ck, Pin, Ckin, Pout, Cko)
    grid = N // ROWB
    return pl.pallas_call(
        body,
        grid=(grid,),
        in_specs=[
            pl.BlockSpec((2, Pa, ROWB, Ck), lambda i: (0, 0, i, 0)),
            pl.BlockSpec((Pin, ROWB, Ckin), lambda i: (0, i, 0)),
            pl.BlockSpec((Pout * Cko, Pa * Ck), lambda i: (0, 0)),
            pl.BlockSpec((Pout * Cko, Pin * Ckin), lambda i: (0, 0)),
            pl.BlockSpec((1, Pout * Cko), lambda i: (0, 0)),
        ],
        out_specs=pl.BlockSpec((Pout, ROWB, Cko), lambda i: (0, i, 0)),
        out_shape=jax.ShapeDtypeStruct((Pout, N, Cko), jnp.float32),
    )


def _lin_tc_body(Pin, Ckin, Pout, Cko, h_ref, w_ref, out_ref):
    for po in range(Pout):
        acc = jnp.zeros((ROWB, Cko), jnp.float32)
        for pi in range(Pin):
            wslice = w_ref[pl.ds(po * Cko, Cko), pl.ds(pi * Ckin, Ckin)]
            acc = acc + _dotT16(h_ref[pi], wslice)
        out_ref[po] = acc


@functools.lru_cache(maxsize=None)
def _make_lin_tc(Pin, Ckin, Pout, Cko):
    body = functools.partial(_lin_tc_body, Pin, Ckin, Pout, Cko)
    grid = N // ROWB
    return pl.pallas_call(
        body,
        grid=(grid,),
        in_specs=[
            pl.BlockSpec((Pin, ROWB, Ckin), lambda i: (0, i, 0)),
            pl.BlockSpec((Pout * Cko, Pin * Ckin), lambda i: (0, 0)),
        ],
        out_specs=pl.BlockSpec((Pout, ROWB, Cko), lambda i: (0, i, 0)),
        out_shape=jax.ShapeDtypeStruct((Pout, N, Cko), jnp.float32),
    )


def _gcnpre_tc_body(Pa, Ck, Pin, Ckin, Pout, Cko,
                    part_ref, h_ref, wroot_ref, b_ref, out_ref):
    for po in range(Pout):
        acc = jnp.broadcast_to(b_ref[0, pl.ds(po * Cko, Cko)], (ROWB, Cko))
        acc = acc + part_ref[0, po] + part_ref[1, po]
        for pi in range(Pin):
            wslice = wroot_ref[pl.ds(po * Cko, Cko), pl.ds(pi * Ckin, Ckin)]
            acc = acc + _dotT16(h_ref[pi], wslice)
        out_ref[po] = jnp.maximum(acc, 0.0)


@functools.lru_cache(maxsize=None)
def _make_gcnpre_tc(Pa, Ck, Pin, Ckin, Pout, Cko):
    body = functools.partial(_gcnpre_tc_body, Pa, Ck, Pin, Ckin, Pout, Cko)
    grid = N // ROWB
    return pl.pallas_call(
        body,
        grid=(grid,),
        in_specs=[
            pl.BlockSpec((2, Pa, ROWB, Ck), lambda i: (0, 0, i, 0)),
            pl.BlockSpec((Pin, ROWB, Ckin), lambda i: (0, i, 0)),
            pl.BlockSpec((Pout * Cko, Pin * Ckin), lambda i: (0, 0)),
            pl.BlockSpec((1, Pout * Cko), lambda i: (0, 0)),
        ],
        out_specs=pl.BlockSpec((Pout, ROWB, Cko), lambda i: (0, i, 0)),
        out_shape=jax.ShapeDtypeStruct((Pout, N, Cko), jnp.float32),
    )


def _pool_mlp_body(mlp_shapes, h_ref, batch_ref, w1, b1, w2, b2, w3, b3,
                   wo, out_ref, acc_ref, cnt_ref):
    i = pl.program_id(0)
    P = h_ref.shape[0]
    hcat = jnp.concatenate([h_ref[p] for p in range(P)], axis=1)
    gids = jax.lax.broadcasted_iota(jnp.int32, (ROWB, NUM_GRAPHS), 1)
    onehot = (batch_ref[...] == gids).astype(jnp.float32)
    psum = lax.dot_general(onehot, hcat, (((0,), (0,)), ((), ())),
                           precision=lax.Precision.HIGHEST,
                           preferred_element_type=jnp.float32)
    pcnt = jnp.sum(onehot, axis=0)[:, None]

    @pl.when(i == 0)
    def _init():
        acc_ref[...] = psum
        cnt_ref[...] = pcnt

    @pl.when(i > 0)
    def _acc():
        acc_ref[...] = acc_ref[...] + psum
        cnt_ref[...] = cnt_ref[...] + pcnt

    @pl.when(i == pl.num_programs(0) - 1)
    def _head():
        g = acc_ref[...] / jnp.maximum(cnt_ref[...], 1.0)
        g = jnp.maximum(_dotT16(g, w1[...]) + b1[0], 0.0)
        g = jnp.maximum(_dotT16(g, w2[...]) + b2[0], 0.0)
        g = jnp.maximum(_dotT16(g, w3[...]) + b3[0], 0.0)
        out_ref[...] = _dotT16(g, wo[...])


@functools.lru_cache(maxsize=None)
def _make_pool_mlp(P, Ck, d1i, d1o, d2o, d3o):
    body = functools.partial(_pool_mlp_body, None)
    grid = N // ROWB
    C = P * Ck
    return pl.pallas_call(
        body,
        grid=(grid,),
        in_specs=[
            pl.BlockSpec((P, ROWB, Ck), lambda i: (0, i, 0)),
            pl.BlockSpec((ROWB, 1), lambda i: (i, 0)),
            pl.BlockSpec((d1o, d1i), lambda i: (0, 0)),
            pl.BlockSpec((1, d1o), lambda i: (0, 0)),
            pl.BlockSpec((d2o, d1o), lambda i: (0, 0)),
            pl.BlockSpec((1, d2o), lambda i: (0, 0)),
            pl.BlockSpec((d3o, d2o), lambda i: (0, 0)),
            pl.BlockSpec((1, d3o), lambda i: (0, 0)),
            pl.BlockSpec((128, d3o), lambda i: (0, 0)),
        ],
        out_specs=pl.BlockSpec((NUM_GRAPHS, 128), lambda i: (0, 0)),
        out_shape=jax.ShapeDtypeStruct((NUM_GRAPHS, 128), jnp.float32),
        scratch_shapes=[
            pltpu.VMEM((NUM_GRAPHS, C), jnp.float32),
            pltpu.VMEM((NUM_GRAPHS, 1), jnp.float32),
        ],
    )


def _padded(c):
    return max(128, c)


def kernel(x, edge_index, batch, edge_attr, gcn_params, mlp_params, out_W, out_b):
    src = edge_index[0]
    dst = edge_index[1]

    xp = jnp.pad(x, ((0, 0), (0, 124)))
    h = xp.reshape(1, N, 128)
    Pin, Ckin = 1, 128

    for li, (W_rel, b_rel, W_root) in enumerate(gcn_params):
        cout, cin = W_rel.shape
        cin_p, cout_p = _padded(cin), _padded(cout)
        W_rel = jnp.pad(W_rel, ((0, cout_p - cout), (0, cin_p - cin)))
        W_root = jnp.pad(W_root, ((0, cout_p - cout), (0, cin_p - cin)))
        b_rel = jnp.pad(b_rel, (0, cout_p - cout))
        Pout, Cko = cout_p // 128, 128

        if cout_p < cin_p:
            g = _make_lin_tc(Pin, Ckin, Pout, Cko)(h, W_rel)
            part = _make_sc_segsum(Pout, Cko)(g, src, dst, edge_attr)
            h = _make_gcnpre_tc(Pout, Cko, Pin, Ckin, Pout, Cko)(
                part, h, W_root, b_rel.reshape(1, -1)
            )
        else:
            Pa, Ck = Pin, Ckin
            part = _make_sc_segsum(Pa, Ck)(h, src, dst, edge_attr)
            h = _make_gcn_tc(Pa, Ck, Pin, Ckin, Pout, Cko)(
                part, h, W_rel, W_root, b_rel.reshape(1, -1)
            )
        Pin, Ckin = Pout, Cko

    (W1, b1), (W2, b2), (W3, b3) = mlp_params
    wo_p = jnp.pad(out_W, ((0, 127), (0, 0)))
    pooled = _make_pool_mlp(Pin, Ckin, W1.shape[1], W1.shape[0],
                            W2.shape[0], W3.shape[0])(
        h, batch.reshape(N, 1).astype(jnp.int32),
        W1, b1.reshape(1, -1), W2, b2.reshape(1, -1),
        W3, b3.reshape(1, -1), wo_p,
    )
    return pooled[:, :1] + out_b
